# baseline (speedup 1.0000x reference)
import functools

import jax
import jax.numpy as jnp
from jax import lax
from jax.experimental import pallas as pl
from jax.experimental.pallas import tpu as pltpu
from jax.experimental.pallas import tpu_sc as plsc

B = 16384
INPUT_DIM = 128
EMBED_DIM = 16

NC = 2
NS = 16
NW = NC * NS
BPW = B // NW
IDX_CHUNK = 128
NCH = BPW // IDX_CHUNK


def _sc_prod_body(table_hbm, idx1_hbm, idx2_hbm, out_hbm,
                  idx1_v, idx2_v, rows1_v, rows2_v, prod_v, sem):
    wid = lax.axis_index("s") * NC + lax.axis_index("c")
    row_base = wid * NCH
    base = wid * BPW

    pltpu.sync_copy(idx1_hbm.at[pl.ds(row_base, NCH)], idx1_v)
    pltpu.sync_copy(idx2_hbm.at[pl.ds(row_base, NCH)], idx2_v)

    cps = []
    for k in range(NCH):
        cps.append(pltpu.async_copy(
            table_hbm.at[idx1_v.at[k]],
            rows1_v.at[pl.ds(k * IDX_CHUNK, IDX_CHUNK)], sem))
        cps.append(pltpu.async_copy(
            table_hbm.at[idx2_v.at[k]],
            rows2_v.at[pl.ds(k * IDX_CHUNK, IDX_CHUNK)], sem))
    for cp in cps:
        cp.wait()

    def body(i, carry):
        prod_v[i, :] = rows1_v[i, :] * rows2_v[i, :]
        return carry

    lax.fori_loop(0, BPW, body, 0)

    pltpu.sync_copy(prod_v, out_hbm.at[pl.ds(base, BPW)])


_sc_prod = functools.partial(
    pl.kernel,
    mesh=plsc.VectorSubcoreMesh(core_axis_name="c", subcore_axis_name="s",
                                num_cores=NC, num_subcores=NS),
    compiler_params=pltpu.CompilerParams(use_tc_tiling_on_sc=False),
    out_type=jax.ShapeDtypeStruct((B, EMBED_DIM), jnp.float32),
    scratch_types=[
        pltpu.VMEM((NCH, IDX_CHUNK), jnp.int32),
        pltpu.VMEM((NCH, IDX_CHUNK), jnp.int32),
        pltpu.VMEM((BPW, EMBED_DIM), jnp.float32),
        pltpu.VMEM((BPW, EMBED_DIM), jnp.float32),
        pltpu.VMEM((BPW, EMBED_DIM), jnp.float32),
        pltpu.SemaphoreType.DMA,
    ],
)(_sc_prod_body)


TC_BLK = 2048
TC_NBLK = B // TC_BLK


def _tc_body(x1_ref, x2_ref, w_ref, prod_ref, scale_ref, out_ref):
    d = x1_ref[...] - x2_ref[...]
    feat = lax.dot_general(w_ref[...], d, (((1,), (1,)), ((), ())),
                           preferred_element_type=jnp.float32)
    ones = jnp.ones((1, EMBED_DIM), jnp.float32)
    inter = lax.dot_general(ones, prod_ref[...], (((1,), (1,)), ((), ())),
                            preferred_element_type=jnp.float32)
    out_ref[...] = (feat + scale_ref[0, 0] * inter)[None]


_tc_combine = pl.pallas_call(
    _tc_body,
    grid=(TC_NBLK,),
    in_specs=[
        pl.BlockSpec((TC_BLK, INPUT_DIM), lambda i: (i, 0)),
        pl.BlockSpec((TC_BLK, INPUT_DIM), lambda i: (i, 0)),
        pl.BlockSpec((1, INPUT_DIM), lambda i: (0, 0)),
        pl.BlockSpec((TC_BLK, EMBED_DIM), lambda i: (i, 0)),
        pl.BlockSpec(memory_space=pltpu.SMEM),
    ],
    out_specs=pl.BlockSpec((1, 1, TC_BLK), lambda i: (i, 0, 0)),
    out_shape=jax.ShapeDtypeStruct((TC_NBLK, 1, TC_BLK), jnp.float32),
)


def kernel(x_1, x_2, idx_1, idx_2, W, table, scale):
    idx1 = idx_1.astype(jnp.int32).reshape(B // IDX_CHUNK, IDX_CHUNK)
    idx2 = idx_2.astype(jnp.int32).reshape(B // IDX_CHUNK, IDX_CHUNK)
    prod = _sc_prod(table, idx1, idx2)
    scale2d = jnp.asarray(scale, jnp.float32).reshape(1, 1)
    out = _tc_combine(x_1, x_2, W, prod, scale2d)
    return out.reshape(B, 1)


# speedup vs baseline: 1.0060x; 1.0060x over previous
import functools

import jax
import jax.numpy as jnp
from jax import lax
from jax.experimental import pallas as pl
from jax.experimental.pallas import tpu as pltpu
from jax.experimental.pallas import tpu_sc as plsc

B = 16384
INPUT_DIM = 128
EMBED_DIM = 16
NUM_CHAMPS = 100000

NC = 2
NS = 16
NW = NC * NS
BPW = B // NW
GCH = 128
NGC = BPW // GCH


def _sc_prod_body(table_hbm, idx1_hbm, idx2_hbm, out_hbm,
                  idx1_v, idx2_v, prod_v,
                  r1a, r1b, r2a, r2b,
                  sem1a, sem1b, sem2a, sem2b):
    wid = lax.axis_index("s") * NC + lax.axis_index("c")

    a = wid // 2
    b0 = 4 * (wid % 2)
    pltpu.sync_copy(idx1_hbm.at[a, pl.ds(b0, NGC)], idx1_v)
    pltpu.sync_copy(idx2_hbm.at[a, pl.ds(b0, NGC)], idx2_v)

    r1 = (r1a, r1b)
    r2 = (r2a, r2b)
    sems1 = (sem1a, sem1b)
    sems2 = (sem2a, sem2b)
    cps = {}

    def fire(k):
        b = k % 2
        cps[(1, k)] = pltpu.async_copy(table_hbm.at[idx1_v.at[k]], r1[b], sems1[b])
        cps[(2, k)] = pltpu.async_copy(table_hbm.at[idx2_v.at[k]], r2[b], sems2[b])

    def consume(k):
        b = k % 2
        cps.pop((1, k)).wait()
        cps.pop((2, k)).wait()
        for n in range(GCH):
            g = k * GCH + n
            e1 = r1[b][n, pl.ds(0, EMBED_DIM)]
            e2 = r2[b][n, pl.ds(0, EMBED_DIM)]
            prod_v[g // 64, (g // 8) % 8, pl.ds(16 * (g % 8), 16)] = e1 * e2

    fire(0)
    fire(1)
    for k in range(NGC):
        consume(k)
        if k + 2 < NGC:
            fire(k + 2)

    pltpu.sync_copy(prod_v, out_hbm.at[pl.ds(8 * wid, 8)])


_sc_prod = functools.partial(
    pl.kernel,
    mesh=plsc.VectorSubcoreMesh(core_axis_name="c", subcore_axis_name="s",
                                num_cores=NC, num_subcores=NS),
    compiler_params=pltpu.CompilerParams(use_tc_tiling_on_sc=False),
    out_type=jax.ShapeDtypeStruct((B // 64, 8, 128), jnp.float32),
    scratch_types=[
        pltpu.VMEM((NGC, GCH), jnp.int32),
        pltpu.VMEM((NGC, GCH), jnp.int32),
        pltpu.VMEM((BPW // 64, 8, 128), jnp.float32),
        pltpu.VMEM((GCH, 128), jnp.float32),
        pltpu.VMEM((GCH, 128), jnp.float32),
        pltpu.VMEM((GCH, 128), jnp.float32),
        pltpu.VMEM((GCH, 128), jnp.float32),
        pltpu.SemaphoreType.DMA,
        pltpu.SemaphoreType.DMA,
        pltpu.SemaphoreType.DMA,
        pltpu.SemaphoreType.DMA,
    ],
)(_sc_prod_body)


TC_BLK = 2048
TC_NBLK = B // TC_BLK


def _tc_body(x1_ref, x2_ref, w_ref, prod_ref, scale_ref, out_ref):
    d = x1_ref[...] - x2_ref[...]
    feat = lax.dot_general(w_ref[...], d, (((1,), (1,)), ((), ())),
                           preferred_element_type=jnp.float32)
    pr = prod_ref[...].reshape(TC_BLK // 8, 128)
    pr8 = jnp.broadcast_to(pr[:, None, :], (TC_BLK // 8, 8, 128))
    pr8 = pr8.reshape(TC_BLK, 128)
    si = lax.broadcasted_iota(jnp.int32, (TC_BLK, 128), 0)
    ci = lax.broadcasted_iota(jnp.int32, (TC_BLK, 128), 1)
    msk = (ci // EMBED_DIM == si % 8).astype(jnp.float32)
    ones = jnp.ones((1, 128), jnp.float32)
    inter = lax.dot_general(ones, pr8 * msk, (((1,), (1,)), ((), ())),
                            preferred_element_type=jnp.float32)
    out_ref[...] = feat + scale_ref[0, 0] * inter


_tc_combine = pl.pallas_call(
    _tc_body,
    grid=(TC_NBLK,),
    in_specs=[
        pl.BlockSpec((TC_BLK, INPUT_DIM), lambda i: (i, 0)),
        pl.BlockSpec((TC_BLK, INPUT_DIM), lambda i: (i, 0)),
        pl.BlockSpec((1, INPUT_DIM), lambda i: (0, 0)),
        pl.BlockSpec((TC_BLK // 64, 8, 128), lambda i: (i, 0, 0)),
        pl.BlockSpec(memory_space=pltpu.SMEM),
    ],
    out_specs=pl.BlockSpec((1, TC_BLK), lambda i: (0, i)),
    out_shape=jax.ShapeDtypeStruct((1, B), jnp.float32),
)


def kernel(x_1, x_2, idx_1, idx_2, W, table, scale):
    tpad = jnp.pad(table, ((0, 0), (0, 128 - EMBED_DIM)))
    idx1 = idx_1.astype(jnp.int32).reshape(B // 1024, 8, 128)
    idx2 = idx_2.astype(jnp.int32).reshape(B // 1024, 8, 128)
    prod = _sc_prod(tpad, idx1, idx2)
    scale2d = jnp.asarray(scale, jnp.float32).reshape(1, 1)
    out = _tc_combine(x_1, x_2, W, prod, scale2d)
    return out.reshape(B, 1)
